# baseline (speedup 1.0000x reference)
import functools

import jax
import jax.numpy as jnp
from jax import lax
from jax.experimental import pallas as pl
from jax.experimental.pallas import tpu as pltpu
from jax.experimental.pallas import tpu_sc as plsc

_NUM_TOKENS = 4096
_TOP_K = 2

_info = plsc.get_sparse_core_info()
_NC, _NS = _info.num_cores, _info.num_subcores
_NW = _NC * _NS
_ROWS_PER_W = _NUM_TOKENS // _NW

_mesh = plsc.VectorSubcoreMesh(core_axis_name="c", subcore_axis_name="s")


@functools.partial(
    pl.kernel,
    out_type=(
        jax.ShapeDtypeStruct((_NUM_TOKENS, _TOP_K), jnp.int32),
        jax.ShapeDtypeStruct((_NUM_TOKENS, _TOP_K), jnp.float32),
    ),
    mesh=_mesh,
    scratch_types=(
        pltpu.SemaphoreType.DMA,
        pltpu.SemaphoreType.DMA,
    ),
)
def _route_copy(rt_hbm, rs_hbm, out_rt, out_rs, sem_rt, sem_rs):
    wid = lax.axis_index("s") * _NC + lax.axis_index("c")

    @pl.when(wid == 0)
    def _():
        c1 = pltpu.make_async_copy(rt_hbm, out_rt, sem_rt)
        c2 = pltpu.make_async_copy(rs_hbm, out_rs, sem_rs)
        c1.start()
        c2.start()
        c1.wait()
        c2.wait()


def kernel(router_logits, routing_tensor, routing_scales):
    del router_logits
    return _route_copy(routing_tensor, routing_scales)


# speedup vs baseline: 1.0005x; 1.0005x over previous
import functools

import jax
import jax.numpy as jnp
from jax import lax
from jax.experimental import pallas as pl
from jax.experimental.pallas import tpu as pltpu
from jax.experimental.pallas import tpu_sc as plsc

_NUM_TOKENS = 4096
_TOP_K = 2

_info = plsc.get_sparse_core_info()
_NC, _NS = _info.num_cores, _info.num_subcores
_NW = _NC * _NS
_ROWS_PER_W = _NUM_TOKENS // _NW

_mesh = plsc.ScalarSubcoreMesh(axis_name="c")

_HALF = _NUM_TOKENS // 2


@functools.partial(
    pl.kernel,
    out_type=(
        jax.ShapeDtypeStruct((_NUM_TOKENS, _TOP_K), jnp.int32),
        jax.ShapeDtypeStruct((_NUM_TOKENS, _TOP_K), jnp.float32),
    ),
    mesh=_mesh,
    scratch_types=(
        pltpu.SemaphoreType.DMA,
        pltpu.SemaphoreType.DMA,
    ),
)
def _route_copy(rt_hbm, rs_hbm, out_rt, out_rs, sem_rt, sem_rs):
    cid = lax.axis_index("c")
    sl = pl.ds(cid * _HALF, _HALF)
    c1 = pltpu.make_async_copy(rt_hbm.at[sl], out_rt.at[sl], sem_rt)
    c2 = pltpu.make_async_copy(rs_hbm.at[sl], out_rs.at[sl], sem_rs)
    c1.start()
    c2.start()
    c1.wait()
    c2.wait()


def kernel(router_logits, routing_tensor, routing_scales):
    del router_logits
    return _route_copy(routing_tensor, routing_scales)


# speedup vs baseline: 10.0916x; 10.0865x over previous
import jax
import jax.numpy as jnp
from jax.experimental import pallas as pl

_NUM_TOKENS = 4096
_TOP_K = 2


def _copy_body(rt_ref, rs_ref, out_rt_ref, out_rs_ref):
    out_rt_ref[...] = rt_ref[...]
    out_rs_ref[...] = rs_ref[...]


_SHAPE2D = (_NUM_TOKENS * _TOP_K // 128, 128)

_copy = pl.pallas_call(
    _copy_body,
    out_shape=(
        jax.ShapeDtypeStruct(_SHAPE2D, jnp.int32),
        jax.ShapeDtypeStruct(_SHAPE2D, jnp.float32),
    ),
)


def kernel(router_logits, routing_tensor, routing_scales):
    del router_logits
    rt, rs = _copy(
        routing_tensor.reshape(_SHAPE2D), routing_scales.reshape(_SHAPE2D)
    )
    return (
        rt.reshape(_NUM_TOKENS, _TOP_K),
        rs.reshape(_NUM_TOKENS, _TOP_K),
    )
